# Initial kernel scaffold; baseline (speedup 1.0000x reference)
import jax
import jax.numpy as jnp
from jax.experimental import pallas as pl


def kernel(embs, edge_index, edge_vals):
    raise NotImplementedError("write your pallas kernel here")


# speedup vs baseline: 2.1144x; 2.1144x over previous
import functools

import jax
import jax.numpy as jnp
from jax import lax
from jax.experimental import pallas as pl
from jax.experimental.pallas import tpu as pltpu
from jax.experimental.pallas import tpu_sc as plsc

_N_REL = 3
_D = 128

_NC = 2
_NS = 16
_NW = _NC * _NS
_BLK = 256
_NBLK = 40
_EPAD = _NW * _NBLK * _BLK


def _sc_aggregate(src, dst, vals, embs):
    n_rel = src.shape[0]
    n_pad = ((embs.shape[0] + 639) // 640) * 640
    rows_per_tile = n_pad // _NS

    mesh = plsc.VectorSubcoreMesh(core_axis_name="c", subcore_axis_name="s")

    @functools.partial(
        pl.kernel,
        mesh=mesh,
        compiler_params=pltpu.CompilerParams(needs_layout_passes=False),
        out_type=jax.ShapeDtypeStruct((n_rel, _NC, n_pad, _D), jnp.float32),
        scratch_types=[
            pltpu.VMEM((2, 128), jnp.int32),
            pltpu.VMEM((2, 128), jnp.int32),
            pltpu.VMEM((_BLK,), jnp.float32),
            pltpu.VMEM((_BLK, _D), jnp.float32),
            pltpu.VMEM_SHARED((n_pad, _D), jnp.float32),
            pltpu.SemaphoreType.DMA,
        ],
    )
    def k(src_hbm, dst_hbm, vals_hbm, embs_hbm, out_hbm,
          src_v, dst_v, val_v, rows_v, acc, sem):
        cid = lax.axis_index("c")
        sid = lax.axis_index("s")
        wid = cid * _NS + sid
        row0 = sid * rows_per_tile

        zero16 = jnp.zeros((16,), jnp.float32)

        def zero_rows_row(i, c):
            for kk in range(_D // 16):
                rows_v[i, pl.ds(kk * 16, 16)] = zero16
            return c

        def scale_edge(e, c):
            vspl = plsc.load_gather(val_v, [jnp.full((16,), e, jnp.int32)])
            for kk in range(_D // 16):
                rows_v[e, pl.ds(kk * 16, 16)] = (
                    rows_v[e, pl.ds(kk * 16, 16)] * vspl)
            return c

        for r in range(n_rel):
            lax.fori_loop(0, _BLK, zero_rows_row, 0)
            for off in range(0, rows_per_tile, _BLK):
                sz = min(_BLK, rows_per_tile - off)
                pltpu.sync_copy(rows_v.at[pl.ds(0, sz)],
                                acc.at[pl.ds(row0 + off, sz)])
            plsc.subcore_barrier()

            def block(b, c, r=r):
                pltpu.sync_copy(src_hbm.at[r, wid, b], src_v)
                pltpu.sync_copy(dst_hbm.at[r, wid, b], dst_v)
                pltpu.sync_copy(vals_hbm.at[r, wid, b], val_v)
                cps = [
                    pltpu.async_copy(
                        embs_hbm.at[src_v.at[j]],
                        rows_v.at[pl.ds(j * 128, 128)], sem)
                    for j in range(_BLK // 128)
                ]
                for cp in cps:
                    cp.wait()
                lax.fori_loop(0, _BLK, scale_edge, 0)
                for j in range(_BLK // 128):
                    pltpu.sync_copy(rows_v.at[pl.ds(j * 128, 128)],
                                    acc.at[dst_v.at[j]], add=True)
                return c

            lax.fori_loop(0, _NBLK, block, 0)
            plsc.subcore_barrier()
            pltpu.sync_copy(acc.at[pl.ds(row0, rows_per_tile)],
                            out_hbm.at[r, cid, pl.ds(row0, rows_per_tile)])

    return k(src, dst, vals, embs)


def _tc_combine(partials, n_nodes):
    n_rel, nc, n_pad, d = partials.shape
    p = partials.reshape(n_rel * nc, n_pad, d)
    br = 1000

    def body(p_ref, o_ref):
        acc = None
        for r in range(n_rel):
            x = p_ref[nc * r]
            for c in range(1, nc):
                x = x + p_ref[nc * r + c]
            y = jnp.maximum(x, 0.01 * x)
            acc = y if acc is None else acc + y
        o_ref[...] = acc

    return pl.pallas_call(
        body,
        grid=(n_nodes // br,),
        in_specs=[pl.BlockSpec((n_rel * nc, br, d), lambda i: (0, i, 0))],
        out_specs=pl.BlockSpec((br, d), lambda i: (i, 0)),
        out_shape=jax.ShapeDtypeStruct((n_nodes, d), jnp.float32),
    )(p)


def kernel(embs, edge_index, edge_vals):
    dst = edge_index[:, 0, :].astype(jnp.int32)
    src = edge_index[:, 1, :].astype(jnp.int32)
    vals = edge_vals.astype(jnp.float32)
    pad = _EPAD - src.shape[1]
    src = jnp.pad(src, ((0, 0), (0, pad))).reshape(_N_REL, _NW, _NBLK, 2, 128)
    dst = jnp.pad(dst, ((0, 0), (0, pad))).reshape(_N_REL, _NW, _NBLK, 2, 128)
    vals = jnp.pad(vals, ((0, 0), (0, pad))).reshape(_N_REL, _NW, _NBLK, _BLK)
    partials = _sc_aggregate(src, dst, vals, embs)
    return _tc_combine(partials, embs.shape[0])


# speedup vs baseline: 2.2609x; 1.0693x over previous
import functools

import jax
import jax.numpy as jnp
from jax import lax
from jax.experimental import pallas as pl
from jax.experimental.pallas import tpu as pltpu
from jax.experimental.pallas import tpu_sc as plsc

_N_REL = 3
_D = 128

_NC = 2
_NS = 16
_NW = _NC * _NS
_BLK = 128
_NBLK = 80
_EPAD = _NW * _NBLK * _BLK


def _sc_aggregate(meta, embs):
    n_rel = meta.shape[0]
    n_pad = ((embs.shape[0] + 639) // 640) * 640
    rows_per_tile = n_pad // _NS

    mesh = plsc.VectorSubcoreMesh(core_axis_name="c", subcore_axis_name="s")

    @functools.partial(
        pl.kernel,
        mesh=mesh,
        compiler_params=pltpu.CompilerParams(needs_layout_passes=False),
        out_type=jax.ShapeDtypeStruct((n_rel, _NC, n_pad, _D), jnp.float32),
        scratch_types=[
            pltpu.VMEM((3, 128), jnp.int32),
            pltpu.VMEM((3, 128), jnp.int32),
            pltpu.VMEM((_BLK, _D), jnp.float32),
            pltpu.VMEM((_BLK, _D), jnp.float32),
            pltpu.VMEM_SHARED((n_pad, _D), jnp.float32),
            pltpu.SemaphoreType.DMA,
            pltpu.SemaphoreType.DMA,
            pltpu.SemaphoreType.DMA,
            pltpu.SemaphoreType.DMA,
        ],
    )
    def k(meta_hbm, embs_hbm, out_hbm,
          meta0, meta1, rows0, rows1, acc, gsem0, gsem1, ssem0, ssem1):
        cid = lax.axis_index("c")
        sid = lax.axis_index("s")
        wid = cid * _NS + sid
        row0 = sid * rows_per_tile

        zero16 = jnp.zeros((16,), jnp.float32)
        two16 = jnp.full((16,), 2, jnp.int32)

        def zero_row(i, c):
            for kk in range(_D // 16):
                rows0[i, pl.ds(kk * 16, 16)] = zero16
            return c

        def scale(rows_ref, meta_ref):
            def grp(gi, c):
                for j in range(16):
                    col = gi * 16 + j
                    vbits = plsc.load_gather(
                        meta_ref, [two16, jnp.full((16,), col, jnp.int32)])
                    vspl = plsc.bitcast(vbits, jnp.float32)
                    for kk in range(_D // 16):
                        rows_ref[col, pl.ds(kk * 16, 16)] = (
                            rows_ref[col, pl.ds(kk * 16, 16)] * vspl)
                return c

            lax.fori_loop(0, _BLK // 16, grp, 0)

        for r in range(n_rel):
            lax.fori_loop(0, _BLK, zero_row, 0)
            for j in range(rows_per_tile // _BLK):
                pltpu.sync_copy(rows0, acc.at[pl.ds(row0 + j * _BLK, _BLK)])
            plsc.subcore_barrier()

            pltpu.sync_copy(meta_hbm.at[r, wid, 0], meta0)
            pltpu.async_copy(embs_hbm.at[meta0.at[0]], rows0, gsem0)

            def pair(g, c, r=r):
                b0 = 2 * g
                pltpu.make_async_copy(
                    embs_hbm.at[meta0.at[0]], rows0, gsem0).wait()
                scale(rows0, meta0)

                @pl.when(g > 0)
                def _():
                    pltpu.make_async_copy(
                        rows1, acc.at[meta1.at[1]], ssem1).wait()

                pltpu.sync_copy(meta_hbm.at[r, wid, b0 + 1], meta1)
                g1 = pltpu.async_copy(embs_hbm.at[meta1.at[0]], rows1, gsem1)
                s0 = pltpu.async_copy(rows0, acc.at[meta0.at[1]], ssem0,
                                      add=True)
                g1.wait()
                scale(rows1, meta1)
                s0.wait()

                @pl.when(b0 + 2 < _NBLK)
                def _():
                    pltpu.sync_copy(meta_hbm.at[r, wid, b0 + 2], meta0)
                    pltpu.async_copy(embs_hbm.at[meta0.at[0]], rows0, gsem0)

                pltpu.async_copy(rows1, acc.at[meta1.at[1]], ssem1, add=True)
                return c

            lax.fori_loop(0, _NBLK // 2, pair, 0)
            pltpu.make_async_copy(rows1, acc.at[meta1.at[1]], ssem1).wait()
            plsc.subcore_barrier()
            pltpu.sync_copy(acc.at[pl.ds(row0, rows_per_tile)],
                            out_hbm.at[r, cid, pl.ds(row0, rows_per_tile)])

    return k(meta, embs)


def _tc_combine(partials, n_nodes):
    n_rel, nc, n_pad, d = partials.shape
    p = partials.reshape(n_rel * nc, n_pad, d)
    br = 1000

    def body(p_ref, o_ref):
        acc = None
        for r in range(n_rel):
            x = p_ref[nc * r]
            for c in range(1, nc):
                x = x + p_ref[nc * r + c]
            y = jnp.maximum(x, 0.01 * x)
            acc = y if acc is None else acc + y
        o_ref[...] = acc

    return pl.pallas_call(
        body,
        grid=(n_nodes // br,),
        in_specs=[pl.BlockSpec((n_rel * nc, br, d), lambda i: (0, i, 0))],
        out_specs=pl.BlockSpec((br, d), lambda i: (i, 0)),
        out_shape=jax.ShapeDtypeStruct((n_nodes, d), jnp.float32),
    )(p)


def kernel(embs, edge_index, edge_vals):
    dst = edge_index[:, 0, :].astype(jnp.int32)
    src = edge_index[:, 1, :].astype(jnp.int32)
    vals = edge_vals.astype(jnp.float32)
    pad = _EPAD - src.shape[1]
    src = jnp.pad(src, ((0, 0), (0, pad))).reshape(_N_REL, _NW, _NBLK, 128)
    dst = jnp.pad(dst, ((0, 0), (0, pad))).reshape(_N_REL, _NW, _NBLK, 128)
    vbits = lax.bitcast_convert_type(
        jnp.pad(vals, ((0, 0), (0, pad))), jnp.int32
    ).reshape(_N_REL, _NW, _NBLK, 128)
    meta = jnp.stack([src, dst, vbits], axis=3)
    partials = _sc_aggregate(meta, embs)
    return _tc_combine(partials, embs.shape[0])
